# baseline (speedup 1.0000x reference)
import jax
import jax.numpy as jnp
from jax.experimental import pallas as pl
from jax.experimental.pallas import tpu as pltpu


def _gate_kernel(x_ref, w_ref, keys_ref, b_ref, o_ref, q_ref, acc_ref):
    j = pl.program_id(1)
    nj = pl.num_programs(1)
    bj = w_ref.shape[0]
    q = jax.lax.dot_general(
        x_ref[...], w_ref[...],
        dimension_numbers=(((1,), (1,)), ((), ())),
        preferred_element_type=jnp.float32) + b_ref[...]
    q_ref[:, pl.ds(j * bj, bj)] = q.astype(jnp.bfloat16)

    @pl.when(j == nj - 1)
    def _finish():
        s = jax.lax.dot_general(
            q_ref[...], keys_ref[...].astype(jnp.bfloat16),
            dimension_numbers=(((1,), (1,)), ((), ())),
            preferred_element_type=jnp.float32)
        s = s - jnp.max(s, axis=1, keepdims=True)
        e = jnp.exp(s)
        o_ref[...] = e / jnp.sum(e, axis=1, keepdims=True)

    del acc_ref


def kernel(x, keys, topk, W, b):
    del topk
    bs, d = x.shape
    ne = keys.shape[0]
    b2 = b.reshape(1, d)

    bm = 1024
    bj = 256
    gates = pl.pallas_call(
        _gate_kernel,
        grid=(bs // bm, d // bj),
        in_specs=[
            pl.BlockSpec((bm, d), lambda i, j: (i, 0)),
            pl.BlockSpec((bj, d), lambda i, j: (j, 0)),
            pl.BlockSpec((ne, d), lambda i, j: (0, 0)),
            pl.BlockSpec((1, bj), lambda i, j: (0, j)),
        ],
        out_specs=pl.BlockSpec((bm, ne), lambda i, j: (i, 0)),
        out_shape=jax.ShapeDtypeStruct((bs, ne), jnp.float32),
        scratch_shapes=[
            pltpu.VMEM((bm, d), jnp.bfloat16),
            pltpu.VMEM((bm, ne), jnp.float32),
        ],
        compiler_params=pltpu.CompilerParams(
            dimension_semantics=("parallel", "arbitrary"),
            vmem_limit_bytes=100 * 1024 * 1024),
    )(x, W, keys, b2)
    return gates


# speedup vs baseline: 1.0323x; 1.0323x over previous
import jax
import jax.numpy as jnp
from jax.experimental import pallas as pl
from jax.experimental.pallas import tpu as pltpu


def _gate_kernel(x_ref, w_ref, keys_ref, b_ref, o_ref, acc_ref):
    j = pl.program_id(1)
    nj = pl.num_programs(1)
    q = jax.lax.dot_general(
        x_ref[...].astype(jnp.bfloat16), w_ref[...],
        dimension_numbers=(((1,), (1,)), ((), ())),
        preferred_element_type=jnp.float32) + b_ref[...]
    part = jax.lax.dot_general(
        q.astype(jnp.bfloat16), keys_ref[...].astype(jnp.bfloat16),
        dimension_numbers=(((1,), (1,)), ((), ())),
        preferred_element_type=jnp.float32)

    @pl.when(j == 0)
    def _init():
        acc_ref[...] = part

    @pl.when(j > 0)
    def _accum():
        acc_ref[...] += part

    @pl.when(j == nj - 1)
    def _finish():
        s = acc_ref[...]
        s = s - jnp.max(s, axis=1, keepdims=True)
        e = jnp.exp(s)
        o_ref[...] = e / jnp.sum(e, axis=1, keepdims=True)


def kernel(x, keys, topk, W, b):
    del topk
    bs, d = x.shape
    ne = keys.shape[0]
    b2 = b.reshape(1, d)
    w16 = W.astype(jnp.bfloat16)

    bm = 1024
    bj = 1024
    gates = pl.pallas_call(
        _gate_kernel,
        grid=(bs // bm, d // bj),
        in_specs=[
            pl.BlockSpec((bm, d), lambda i, j: (i, 0)),
            pl.BlockSpec((bj, d), lambda i, j: (j, 0)),
            pl.BlockSpec((ne, bj), lambda i, j: (0, j)),
            pl.BlockSpec((1, bj), lambda i, j: (0, j)),
        ],
        out_specs=pl.BlockSpec((bm, ne), lambda i, j: (i, 0)),
        out_shape=jax.ShapeDtypeStruct((bs, ne), jnp.float32),
        scratch_shapes=[pltpu.VMEM((bm, ne), jnp.float32)],
        compiler_params=pltpu.CompilerParams(
            dimension_semantics=("parallel", "arbitrary"),
            vmem_limit_bytes=100 * 1024 * 1024),
    )(x, w16, keys, b2)
    return gates
